# baseline (speedup 1.0000x reference)
import functools

import jax
import jax.numpy as jnp
from jax import lax
from jax.experimental import pallas as pl
from jax.experimental.pallas import tpu as pltpu
from jax.experimental.pallas import tpu_sc as plsc

_B, _C, _H, _W = 8, 3, 384, 384
_N = _H * _W
_K = 1327
_PAD = 3
_CLIP = 0.89

_NC, _NS, _L = 2, 16, 16
_GPI = 4
_CHUNK = _N // _GPI
_VPT = _CHUNK // _L
_LEVELS = ((23, 8), (15, 8), (7, 8), (0, 7))


def _dark_channel(x):
    cmin = jnp.min(x, axis=0)
    inf_rows = jnp.full((_PAD, _W), jnp.inf, dtype=cmin.dtype)
    padv = jnp.concatenate([inf_rows, cmin, inf_rows], axis=0)
    vmin = padv[0:_H]
    for dy in range(1, 2 * _PAD + 1):
        vmin = jnp.minimum(vmin, padv[dy:dy + _H])
    inf_cols = jnp.full((_H, _PAD), jnp.inf, dtype=cmin.dtype)
    padh = jnp.concatenate([inf_cols, vmin, inf_cols], axis=1)
    hmin = padh[:, 0:_W]
    for dx in range(1, 2 * _PAD + 1):
        hmin = jnp.minimum(hmin, padh[:, dx:dx + _W])
    return hmin


def _dc_kernel(img_ref, dc_ref):
    dc_ref[0] = lax.bitcast_convert_type(_dark_channel(img_ref[0]), jnp.int32)


def _sc_select_body(dc_hbm, img_hbm, out_hbm,
                    dc_v, img_v, sel_v, hist_v, merged_v, tmp4_v,
                    row_v, cnt_v, hist_sh, cnt_sh):
    cid = lax.axis_index("c")
    sid = lax.axis_index("s")
    img_local = sid // _GPI
    image_id = cid * (_NS // _GPI) + img_local
    g = sid % _GPI
    s0 = img_local * _GPI
    base = g * _CHUNK

    lane = lax.iota(jnp.int32, 16)
    ones = jnp.ones((_L,), jnp.int32)
    zeros16 = jnp.zeros((_L,), jnp.int32)

    pltpu.sync_copy(dc_hbm.at[image_id, pl.ds(base, _CHUNK)], dc_v)

    k_l = jnp.int32(_K)
    prefix = jnp.int32(0)
    for (shift, bits) in _LEVELS:
        width = 1 << bits

        def zero_body(j, _):
            hist_v[pl.ds(j * _L, _L)] = zeros16
            return 0
        lax.fori_loop(0, (_L * width) // _L, zero_body, 0)

        psh = shift + bits

        def hist_body(i, _):
            b = dc_v[pl.ds(i * _L, _L)]
            binv = lax.shift_right_logical(b, shift) & (width - 1)
            idx = lane * width + binv
            if psh >= 31:
                plsc.addupdate_scatter(hist_v, [idx], ones)
            else:
                msk = lax.shift_right_logical(b, psh) == prefix
                plsc.addupdate_scatter(hist_v, [idx], ones, mask=msk)
            return 0
        lax.fori_loop(0, _VPT, hist_body, 0)

        def lmerge_body(j, _):
            acc = zeros16
            for l in range(_L):
                acc = acc + hist_v[pl.ds(l * width + j * _L, _L)]
            merged_v[pl.ds(j * _L, _L)] = acc
            return 0
        lax.fori_loop(0, width // _L, lmerge_body, 0)

        pltpu.sync_copy(merged_v.at[pl.ds(0, width)],
                        hist_sh.at[sid, pl.ds(0, width)])
        plsc.subcore_barrier()
        for gg in range(_GPI):
            pltpu.sync_copy(hist_sh.at[s0 + gg, pl.ds(0, width)],
                            tmp4_v.at[pl.ds(gg * width, width)])
        plsc.subcore_barrier()

        def gmerge_body(j, _):
            acc = zeros16
            for gg in range(_GPI):
                acc = acc + tmp4_v[pl.ds(gg * width + j * _L, _L)]
            merged_v[pl.ds(j * _L, _L)] = acc
            return 0
        lax.fori_loop(0, width // _L, gmerge_body, 0)

        nch = width // _L

        def scan_body(j, carry):
            above, nsat = carry
            jc = nch - 1 - j
            h = merged_v[pl.ds(jc * _L, _L)]
            suf = lax.rev(plsc.cumsum(lax.rev(h, (0,))), (0,)) + above
            nsat = nsat + jnp.where(suf >= k_l, 1, 0)
            above = above + jnp.sum(h)
            return (above, nsat)
        _, nsat_vec = lax.fori_loop(0, nch, scan_body,
                                    (jnp.int32(0), zeros16))
        b_l = jnp.sum(nsat_vec) - 1

        def above_body(j, acc):
            h = merged_v[pl.ds(j * _L, _L)]
            bin_ids = j * _L + lane
            return acc + jnp.where(bin_ids > b_l, h, 0)
        acc_vec = lax.fori_loop(0, nch, above_body, zeros16)
        a_l = jnp.sum(acc_vec)

        k_l = k_l - a_l
        prefix = (prefix << bits) | b_l

    t_bits = prefix
    m = k_l

    def tcnt_body(i, acc):
        return acc + jnp.where(dc_v[pl.ds(i * _L, _L)] == t_bits, 1, 0)
    my_ties = jnp.sum(lax.fori_loop(0, _VPT, tcnt_body, zeros16))

    cnt_v[...] = jnp.full((_L,), my_ties, jnp.int32)
    pltpu.sync_copy(cnt_v, cnt_sh.at[sid])
    plsc.subcore_barrier()
    prior = jnp.int32(0)
    for gg in range(_GPI):
        pltpu.sync_copy(cnt_sh.at[s0 + gg], cnt_v)
        c_gg = jnp.max(cnt_v[...])
        prior = prior + jnp.where(jnp.int32(gg) < g, c_gg, 0)
    plsc.subcore_barrier()
    m_g = m - prior

    def sel_body(i, run):
        b = dc_v[pl.ds(i * _L, _L)]
        gt = b > t_bits
        eq = b == t_bits
        pc = plsc.cumsum(jnp.where(eq, 1, 0))
        sel = gt | (eq & ((run + pc) <= m_g))
        sel_v[pl.ds(i * _L, _L)] = jnp.where(sel, 1, 0)
        return run + jnp.max(pc)
    lax.fori_loop(0, _VPT, sel_body, jnp.int32(0))

    for c in range(_C):
        pltpu.sync_copy(img_hbm.at[image_id * _C + c, pl.ds(base, _CHUNK)],
                        img_v)

        def max_body(i, acc):
            keep = sel_v[pl.ds(i * _L, _L)] > 0
            return jnp.maximum(acc, jnp.where(keep, img_v[pl.ds(i * _L, _L)],
                                              -1.0))
        acc = lax.fori_loop(0, _VPT, max_body, jnp.full((_L,), -1.0,
                                                        jnp.float32))
        row_v[pl.ds(c * _L, _L)] = acc

    off = image_id * (_GPI * _C * _L) + g * (_C * _L)
    pltpu.sync_copy(row_v, out_hbm.at[pl.ds(off, _C * _L)])


_sc_select = functools.partial(
    pl.kernel,
    out_type=jax.ShapeDtypeStruct((_B * _GPI * _C * _L,), jnp.float32),
    mesh=plsc.VectorSubcoreMesh(core_axis_name="c", subcore_axis_name="s",
                                num_cores=_NC, num_subcores=_NS),
    compiler_params=pltpu.CompilerParams(needs_layout_passes=False),
    scratch_types=[
        pltpu.VMEM((_CHUNK,), jnp.int32),
        pltpu.VMEM((_CHUNK,), jnp.float32),
        pltpu.VMEM((_CHUNK,), jnp.int32),
        pltpu.VMEM((_L * 256,), jnp.int32),
        pltpu.VMEM((256,), jnp.int32),
        pltpu.VMEM((_GPI * 256,), jnp.int32),
        pltpu.VMEM((_C * _L,), jnp.float32),
        pltpu.VMEM((_L,), jnp.int32),
        pltpu.VMEM_SHARED((_NS, 256), jnp.int32),
        pltpu.VMEM_SHARED((_NS, _L), jnp.int32),
    ],
)(_sc_select_body)


def _combine_kernel(part_ref, out_ref):
    x = part_ref[...]
    total = jnp.float32(0.0)
    for c in range(_C):
        mc = jnp.full((_B, 1), -1.0, jnp.float32)
        for g in range(_GPI):
            blk = x[:, g * _C * _L + c * _L: g * _C * _L + (c + 1) * _L]
            mc = jnp.maximum(mc, jnp.max(blk, axis=1, keepdims=True))
        total = total + jnp.sum(jnp.minimum(mc, _CLIP))
    out_ref[...] = (total / (_B * _C))[None, None]


@jax.jit
def kernel(image):
    dc = pl.pallas_call(
        _dc_kernel,
        grid=(_B,),
        in_specs=[pl.BlockSpec((1, _C, _H, _W), lambda b: (b, 0, 0, 0))],
        out_specs=pl.BlockSpec((1, _H, _W), lambda b: (b, 0, 0)),
        out_shape=jax.ShapeDtypeStruct((_B, _H, _W), jnp.int32),
        compiler_params=pltpu.CompilerParams(
            dimension_semantics=("arbitrary",),
        ),
    )(image)
    partial = _sc_select(dc.reshape(_B, _N), image.reshape(_B * _C, _N))
    out = pl.pallas_call(
        _combine_kernel,
        out_shape=jax.ShapeDtypeStruct((1, 1), jnp.float32),
    )(partial.reshape(_B, _GPI * _C * _L))
    return out.reshape(())


# speedup vs baseline: 1.1721x; 1.1721x over previous
import functools

import jax
import jax.numpy as jnp
from jax import lax
from jax.experimental import pallas as pl
from jax.experimental.pallas import tpu as pltpu
from jax.experimental.pallas import tpu_sc as plsc

_B, _C, _H, _W = 8, 3, 384, 384
_N = _H * _W
_K = 1327
_PAD = 3
_CLIP = 0.89

_NC, _NS, _L = 2, 16, 16
_GPI = 4
_CHUNK = _N // _GPI
_VPT = _CHUNK // _L
_LEVELS = ((23, 8), (15, 8), (7, 8), (0, 7))
_NSUB = 3


def _dark_channel(x):
    cmin = jnp.min(x, axis=0)
    inf_rows = jnp.full((_PAD, _W), jnp.inf, dtype=cmin.dtype)
    padv = jnp.concatenate([inf_rows, cmin, inf_rows], axis=0)
    vmin = padv[0:_H]
    for dy in range(1, 2 * _PAD + 1):
        vmin = jnp.minimum(vmin, padv[dy:dy + _H])
    inf_cols = jnp.full((_H, _PAD), jnp.inf, dtype=cmin.dtype)
    padh = jnp.concatenate([inf_cols, vmin, inf_cols], axis=1)
    hmin = padh[:, 0:_W]
    for dx in range(1, 2 * _PAD + 1):
        hmin = jnp.minimum(hmin, padh[:, dx:dx + _W])
    return hmin


def _dc_kernel(img_ref, dc_ref):
    dc_ref[0] = lax.bitcast_convert_type(_dark_channel(img_ref[0]), jnp.int32)


def _sc_select_body(dc_hbm, img_hbm, out_hbm,
                    dc_v, img_v, sel_v, hist_v, merged_v, tmp4_v,
                    row_v, cnt_v, hist_sh, cnt_sh):
    cid = lax.axis_index("c")
    sid = lax.axis_index("s")
    img_local = sid // _GPI
    image_id = cid * (_NS // _GPI) + img_local
    g = sid % _GPI
    s0 = img_local * _GPI
    base = g * _CHUNK

    lane = lax.iota(jnp.int32, 16)
    ones = jnp.ones((_L,), jnp.int32)
    zeros16 = jnp.zeros((_L,), jnp.int32)

    pltpu.sync_copy(dc_hbm.at[image_id, pl.ds(base, _CHUNK)], dc_v)

    k_l = jnp.int32(_K)
    prefix = jnp.int32(0)
    b_l = jnp.int32(0)
    for lvl, (shift, bits) in enumerate(_LEVELS):
        width = 1 << bits

        def zero_body(j, _):
            hist_v[pl.ds(j * _L, _L)] = zeros16
            return 0
        lax.fori_loop(0, (_L * width) // _L, zero_body, 0)

        psh = shift + bits

        def hist_body(i, _):
            b = dc_v[pl.ds(i * _L, _L)]
            binv = lax.shift_right_logical(b, shift) & (width - 1)
            idx = lane * width + binv
            if psh >= 31:
                plsc.addupdate_scatter(hist_v, [idx], ones)
            else:
                msk = lax.shift_right_logical(b, psh) == prefix
                plsc.addupdate_scatter(hist_v, [idx], ones, mask=msk)
            return 0
        lax.fori_loop(0, _VPT, hist_body, 0)

        def lmerge_body(j, _):
            acc = zeros16
            for l in range(_L):
                acc = acc + hist_v[pl.ds(l * width + j * _L, _L)]
            merged_v[pl.ds(j * _L, _L)] = acc
            return 0
        lax.fori_loop(0, width // _L, lmerge_body, 0)

        pltpu.sync_copy(merged_v.at[pl.ds(0, width)],
                        hist_sh.at[sid, pl.ds(lvl * 256, width)])
        plsc.subcore_barrier()
        for gg in range(_GPI):
            pltpu.sync_copy(hist_sh.at[s0 + gg, pl.ds(lvl * 256, width)],
                            tmp4_v.at[pl.ds(gg * width, width)])

        def gmerge_body(j, _):
            acc = zeros16
            for gg in range(_GPI):
                acc = acc + tmp4_v[pl.ds(gg * width + j * _L, _L)]
            merged_v[pl.ds(j * _L, _L)] = acc
            return 0
        lax.fori_loop(0, width // _L, gmerge_body, 0)

        nch = width // _L

        def scan_body(j, carry):
            above, nsat = carry
            jc = nch - 1 - j
            h = merged_v[pl.ds(jc * _L, _L)]
            suf = lax.rev(plsc.cumsum(lax.rev(h, (0,))), (0,)) + above
            nsat = nsat + jnp.where(suf >= k_l, 1, 0)
            above = above + jnp.sum(h)
            return (above, nsat)
        _, nsat_vec = lax.fori_loop(0, nch, scan_body,
                                    (jnp.int32(0), zeros16))
        b_l = jnp.sum(nsat_vec) - 1

        def above_body(j, acc):
            h = merged_v[pl.ds(j * _L, _L)]
            bin_ids = j * _L + lane
            return acc + jnp.where(bin_ids > b_l, h, 0)
        acc_vec = lax.fori_loop(0, nch, above_body, zeros16)
        a_l = jnp.sum(acc_vec)

        k_l = k_l - a_l
        prefix = (prefix << bits) | b_l

    t_bits = prefix
    m = k_l

    last_w = 1 << _LEVELS[-1][1]
    my_ties = jnp.sum(plsc.load_gather(hist_v, [lane * last_w + b_l]))

    cnt_v[...] = jnp.full((_L,), my_ties, jnp.int32)
    pltpu.sync_copy(cnt_v, cnt_sh.at[sid])
    plsc.subcore_barrier()
    prior = jnp.int32(0)
    for gg in range(_GPI):
        pltpu.sync_copy(cnt_sh.at[s0 + gg], cnt_v)
        c_gg = jnp.max(cnt_v[...])
        prior = prior + jnp.where(jnp.int32(gg) < g, c_gg, 0)
    m_g = m - prior

    sub = _CHUNK // _NSUB
    acc0 = jnp.full((_L,), -1.0, jnp.float32)
    run = zeros16
    accs = (acc0, acc0, acc0)
    for sch in range(_NSUB):
        for c in range(_C):
            pltpu.sync_copy(
                img_hbm.at[image_id * _C + c, pl.ds(base + sch * sub, sub)],
                img_v.at[pl.ds(c * sub, sub)])

        def fuse_body(i, carry):
            run, a0, a1, a2 = carry
            b = dc_v[pl.ds(sch * sub + i * _L, _L)]
            gt = b > t_bits
            eq = b == t_bits
            pc = plsc.cumsum(jnp.where(eq, 1, 0))
            sel = gt | (eq & ((run + pc) <= m_g))
            run = run + plsc.all_reduce_population_count(eq)
            v0 = img_v[pl.ds(0 * sub + i * _L, _L)]
            v1 = img_v[pl.ds(1 * sub + i * _L, _L)]
            v2 = img_v[pl.ds(2 * sub + i * _L, _L)]
            a0 = jnp.maximum(a0, jnp.where(sel, v0, -1.0))
            a1 = jnp.maximum(a1, jnp.where(sel, v1, -1.0))
            a2 = jnp.maximum(a2, jnp.where(sel, v2, -1.0))
            return (run, a0, a1, a2)
        run, *accs = lax.fori_loop(0, sub // _L, fuse_body, (run, *accs))

    for c in range(_C):
        row_v[pl.ds(c * _L, _L)] = accs[c]

    off = image_id * (_GPI * _C * _L) + g * (_C * _L)
    pltpu.sync_copy(row_v, out_hbm.at[pl.ds(off, _C * _L)])


_sc_select = functools.partial(
    pl.kernel,
    out_type=jax.ShapeDtypeStruct((_B * _GPI * _C * _L,), jnp.float32),
    mesh=plsc.VectorSubcoreMesh(core_axis_name="c", subcore_axis_name="s",
                                num_cores=_NC, num_subcores=_NS),
    compiler_params=pltpu.CompilerParams(needs_layout_passes=False),
    scratch_types=[
        pltpu.VMEM((_CHUNK,), jnp.int32),
        pltpu.VMEM((_CHUNK,), jnp.float32),
        pltpu.VMEM((_CHUNK,), jnp.int32),
        pltpu.VMEM((_L * 256,), jnp.int32),
        pltpu.VMEM((256,), jnp.int32),
        pltpu.VMEM((_GPI * 256,), jnp.int32),
        pltpu.VMEM((_C * _L,), jnp.float32),
        pltpu.VMEM((_L,), jnp.int32),
        pltpu.VMEM_SHARED((_NS, 4 * 256), jnp.int32),
        pltpu.VMEM_SHARED((_NS, _L), jnp.int32),
    ],
)(_sc_select_body)


def _combine_kernel(part_ref, out_ref):
    x = part_ref[...]
    total = jnp.float32(0.0)
    for c in range(_C):
        mc = jnp.full((_B, 1), -1.0, jnp.float32)
        for g in range(_GPI):
            blk = x[:, g * _C * _L + c * _L: g * _C * _L + (c + 1) * _L]
            mc = jnp.maximum(mc, jnp.max(blk, axis=1, keepdims=True))
        total = total + jnp.sum(jnp.minimum(mc, _CLIP))
    out_ref[...] = (total / (_B * _C))[None, None]


@jax.jit
def kernel(image):
    dc = pl.pallas_call(
        _dc_kernel,
        grid=(_B,),
        in_specs=[pl.BlockSpec((1, _C, _H, _W), lambda b: (b, 0, 0, 0))],
        out_specs=pl.BlockSpec((1, _H, _W), lambda b: (b, 0, 0)),
        out_shape=jax.ShapeDtypeStruct((_B, _H, _W), jnp.int32),
        compiler_params=pltpu.CompilerParams(
            dimension_semantics=("arbitrary",),
        ),
    )(image)
    partial = _sc_select(dc.reshape(_B, _N), image.reshape(_B * _C, _N))
    out = pl.pallas_call(
        _combine_kernel,
        out_shape=jax.ShapeDtypeStruct((1, 1), jnp.float32),
    )(partial.reshape(_B, _GPI * _C * _L))
    return out.reshape(())


# speedup vs baseline: 1.4046x; 1.1983x over previous
import functools

import jax
import jax.numpy as jnp
from jax import lax
from jax.experimental import pallas as pl
from jax.experimental.pallas import tpu as pltpu
from jax.experimental.pallas import tpu_sc as plsc

_B, _C, _H, _W = 8, 3, 384, 384
_N = _H * _W
_K = 1327
_PAD = 3
_CLIP = 0.89

_NC, _NS, _L = 2, 16, 16
_GPI = 4
_CHUNK = _N // _GPI
_VPT = _CHUNK // _L
_LEVELS = ((23, 8), (15, 8), (7, 8), (0, 7))
_NSUB = 3


def _dark_channel(x):
    cmin = jnp.min(x, axis=0)
    inf_rows = jnp.full((_PAD, _W), jnp.inf, dtype=cmin.dtype)
    padv = jnp.concatenate([inf_rows, cmin, inf_rows], axis=0)
    vmin = padv[0:_H]
    for dy in range(1, 2 * _PAD + 1):
        vmin = jnp.minimum(vmin, padv[dy:dy + _H])
    inf_cols = jnp.full((_H, _PAD), jnp.inf, dtype=cmin.dtype)
    padh = jnp.concatenate([inf_cols, vmin, inf_cols], axis=1)
    hmin = padh[:, 0:_W]
    for dx in range(1, 2 * _PAD + 1):
        hmin = jnp.minimum(hmin, padh[:, dx:dx + _W])
    return hmin


def _dc_kernel(img_ref, dc_ref):
    dc_ref[0] = lax.bitcast_convert_type(_dark_channel(img_ref[0]), jnp.int32)


def _sc_select_body(dc_hbm, img_hbm, out_hbm,
                    dc_v, img_v, cand_v, hist_v, merged_v, tmp4_v,
                    row_v, cnt_v, hist_sh, cnt_sh):
    cid = lax.axis_index("c")
    sid = lax.axis_index("s")
    img_local = sid // _GPI
    image_id = cid * (_NS // _GPI) + img_local
    g = sid % _GPI
    s0 = img_local * _GPI
    base = g * _CHUNK

    lane = lax.iota(jnp.int32, 16)
    ones = jnp.ones((_L,), jnp.int32)
    zeros16 = jnp.zeros((_L,), jnp.int32)

    pltpu.sync_copy(dc_hbm.at[image_id, pl.ds(base, _CHUNK)], dc_v)

    k_l = jnp.int32(_K)
    prefix = jnp.int32(0)
    b_l = jnp.int32(0)
    ncand = jnp.int32(0)
    for lvl, (shift, bits) in enumerate(_LEVELS):
        width = 1 << bits

        def zero_body(j, _):
            for u in range(8):
                hist_v[pl.ds((j * 8 + u) * _L, _L)] = zeros16
            return 0
        lax.fori_loop(0, width // 8, zero_body, 0)

        psh = shift + bits

        if lvl == 0:
            def hist0_body(i, _):
                for u in range(8):
                    b = dc_v[pl.ds((i * 8 + u) * _L, _L)]
                    binv = lax.shift_right_logical(b, shift)
                    plsc.addupdate_scatter(hist_v, [lane * width + binv],
                                           ones)
                return 0
            lax.fori_loop(0, _VPT // 8, hist0_body, 0)
        else:
            ncv = (ncand + (_L - 1)) // _L

            def histc_body(i, _):
                b = cand_v[pl.ds(i * _L, _L)]
                binv = lax.shift_right_logical(b, shift) & (width - 1)
                valid = (i * _L + lane) < ncand
                msk = (lax.shift_right_logical(b, psh) == prefix) & valid
                plsc.addupdate_scatter(hist_v, [lane * width + binv], ones,
                                       mask=msk)
                return 0
            lax.fori_loop(0, ncv, histc_body, 0)

        def lmerge_body(j, _):
            acc = zeros16
            for l in range(_L):
                acc = acc + hist_v[pl.ds(l * width + j * _L, _L)]
            merged_v[pl.ds(j * _L, _L)] = acc
            return 0
        lax.fori_loop(0, width // _L, lmerge_body, 0)

        pltpu.sync_copy(merged_v.at[pl.ds(0, width)],
                        hist_sh.at[sid, pl.ds(lvl * 256, width)])
        plsc.subcore_barrier()
        for gg in range(_GPI):
            pltpu.sync_copy(hist_sh.at[s0 + gg, pl.ds(lvl * 256, width)],
                            tmp4_v.at[pl.ds(gg * width, width)])

        def gmerge_body(j, _):
            acc = zeros16
            for gg in range(_GPI):
                acc = acc + tmp4_v[pl.ds(gg * width + j * _L, _L)]
            merged_v[pl.ds(j * _L, _L)] = acc
            return 0
        lax.fori_loop(0, width // _L, gmerge_body, 0)

        nch = width // _L

        def scan_body(j, carry):
            above, nsat = carry
            jc = nch - 1 - j
            h = merged_v[pl.ds(jc * _L, _L)]
            suf = lax.rev(plsc.cumsum(lax.rev(h, (0,))), (0,)) + above
            nsat = nsat + jnp.where(suf >= k_l, 1, 0)
            above = above + jnp.sum(h)
            return (above, nsat)
        _, nsat_vec = lax.fori_loop(0, nch, scan_body,
                                    (jnp.int32(0), zeros16))
        b_l = jnp.sum(nsat_vec) - 1

        def above_body(j, acc):
            h = merged_v[pl.ds(j * _L, _L)]
            bin_ids = j * _L + lane
            return acc + jnp.where(bin_ids > b_l, h, 0)
        acc_vec = lax.fori_loop(0, nch, above_body, zeros16)
        a_l = jnp.sum(acc_vec)

        k_l = k_l - a_l
        prefix = (prefix << bits) | b_l

        if lvl == 0:
            sh0 = _LEVELS[0][0]

            def compact_body(i, ptr):
                for u in range(8):
                    b = dc_v[pl.ds((i * 8 + u) * _L, _L)]
                    eqb = lax.shift_right_logical(b, sh0) == b_l
                    pc = plsc.cumsum(jnp.where(eqb, 1, 0))
                    idx = ptr + jnp.maximum(pc, 1) - 1
                    plsc.store_scatter(cand_v, [idx], b, mask=eqb)
                    ptr = ptr + plsc.all_reduce_population_count(eqb)
                return ptr
            ptr_vec = lax.fori_loop(0, _VPT // 8, compact_body, zeros16)
            ncand = jnp.max(ptr_vec)

    t_bits = prefix
    m = k_l

    last_w = 1 << _LEVELS[-1][1]
    my_ties = jnp.sum(plsc.load_gather(hist_v, [lane * last_w + b_l]))

    cnt_v[...] = jnp.full((_L,), my_ties, jnp.int32)
    pltpu.sync_copy(cnt_v, cnt_sh.at[sid])
    plsc.subcore_barrier()
    prior = jnp.int32(0)
    for gg in range(_GPI):
        pltpu.sync_copy(cnt_sh.at[s0 + gg], cnt_v)
        c_gg = jnp.max(cnt_v[...])
        prior = prior + jnp.where(jnp.int32(gg) < g, c_gg, 0)
    m_g = m - prior

    sub = _CHUNK // _NSUB
    acc0 = jnp.full((_L,), -1.0, jnp.float32)
    run = zeros16
    accs = (acc0, acc0, acc0)
    for sch in range(_NSUB):
        for c in range(_C):
            pltpu.sync_copy(
                img_hbm.at[image_id * _C + c, pl.ds(base + sch * sub, sub)],
                img_v.at[pl.ds(c * sub, sub)])

        def fuse_body(i, carry):
            run, a0, a1, a2 = carry
            for u in range(4):
                iv = i * 4 + u
                b = dc_v[pl.ds(sch * sub + iv * _L, _L)]
                gt = b > t_bits
                eq = b == t_bits
                pc = plsc.cumsum(jnp.where(eq, 1, 0))
                sel = gt | (eq & ((run + pc) <= m_g))
                run = run + plsc.all_reduce_population_count(eq)
                v0 = img_v[pl.ds(0 * sub + iv * _L, _L)]
                v1 = img_v[pl.ds(1 * sub + iv * _L, _L)]
                v2 = img_v[pl.ds(2 * sub + iv * _L, _L)]
                a0 = jnp.maximum(a0, jnp.where(sel, v0, -1.0))
                a1 = jnp.maximum(a1, jnp.where(sel, v1, -1.0))
                a2 = jnp.maximum(a2, jnp.where(sel, v2, -1.0))
            return (run, a0, a1, a2)
        run, *accs = lax.fori_loop(0, sub // (_L * 4), fuse_body,
                                   (run, *accs))

    for c in range(_C):
        row_v[pl.ds(c * _L, _L)] = accs[c]

    off = image_id * (_GPI * _C * _L) + g * (_C * _L)
    pltpu.sync_copy(row_v, out_hbm.at[pl.ds(off, _C * _L)])


_sc_select = functools.partial(
    pl.kernel,
    out_type=jax.ShapeDtypeStruct((_B * _GPI * _C * _L,), jnp.float32),
    mesh=plsc.VectorSubcoreMesh(core_axis_name="c", subcore_axis_name="s",
                                num_cores=_NC, num_subcores=_NS),
    compiler_params=pltpu.CompilerParams(needs_layout_passes=False),
    scratch_types=[
        pltpu.VMEM((_CHUNK,), jnp.int32),
        pltpu.VMEM((_CHUNK,), jnp.float32),
        pltpu.VMEM((_CHUNK,), jnp.int32),
        pltpu.VMEM((_L * 256,), jnp.int32),
        pltpu.VMEM((256,), jnp.int32),
        pltpu.VMEM((_GPI * 256,), jnp.int32),
        pltpu.VMEM((_C * _L,), jnp.float32),
        pltpu.VMEM((_L,), jnp.int32),
        pltpu.VMEM_SHARED((_NS, 4 * 256), jnp.int32),
        pltpu.VMEM_SHARED((_NS, _L), jnp.int32),
    ],
)(_sc_select_body)


def _combine_kernel(part_ref, out_ref):
    x = part_ref[...]
    total = jnp.float32(0.0)
    for c in range(_C):
        mc = jnp.full((_B, 1), -1.0, jnp.float32)
        for g in range(_GPI):
            blk = x[:, g * _C * _L + c * _L: g * _C * _L + (c + 1) * _L]
            mc = jnp.maximum(mc, jnp.max(blk, axis=1, keepdims=True))
        total = total + jnp.sum(jnp.minimum(mc, _CLIP))
    out_ref[...] = (total / (_B * _C))[None, None]


@jax.jit
def kernel(image):
    dc = pl.pallas_call(
        _dc_kernel,
        grid=(_B,),
        in_specs=[pl.BlockSpec((1, _C, _H, _W), lambda b: (b, 0, 0, 0))],
        out_specs=pl.BlockSpec((1, _H, _W), lambda b: (b, 0, 0)),
        out_shape=jax.ShapeDtypeStruct((_B, _H, _W), jnp.int32),
        compiler_params=pltpu.CompilerParams(
            dimension_semantics=("arbitrary",),
        ),
    )(image)
    partial = _sc_select(dc.reshape(_B, _N), image.reshape(_B * _C, _N))
    out = pl.pallas_call(
        _combine_kernel,
        out_shape=jax.ShapeDtypeStruct((1, 1), jnp.float32),
    )(partial.reshape(_B, _GPI * _C * _L))
    return out.reshape(())


# speedup vs baseline: 1.5260x; 1.0864x over previous
import functools

import jax
import jax.numpy as jnp
from jax import lax
from jax.experimental import pallas as pl
from jax.experimental.pallas import tpu as pltpu
from jax.experimental.pallas import tpu_sc as plsc

_B, _C, _H, _W = 8, 3, 384, 384
_N = _H * _W
_K = 1327
_PAD = 3
_CLIP = 0.89

_NC, _NS, _L = 2, 16, 16
_GPI = 4
_CHUNK = _N // _GPI
_VPT = _CHUNK // _L
_LEVELS = ((23, 8), (15, 8), (7, 8), (0, 7))
_NSUB = 3


def _dark_channel(x):
    cmin = jnp.min(x, axis=0)
    inf_rows = jnp.full((_PAD, _W), jnp.inf, dtype=cmin.dtype)
    padv = jnp.concatenate([inf_rows, cmin, inf_rows], axis=0)
    vmin = padv[0:_H]
    for dy in range(1, 2 * _PAD + 1):
        vmin = jnp.minimum(vmin, padv[dy:dy + _H])
    inf_cols = jnp.full((_H, _PAD), jnp.inf, dtype=cmin.dtype)
    padh = jnp.concatenate([inf_cols, vmin, inf_cols], axis=1)
    hmin = padh[:, 0:_W]
    for dx in range(1, 2 * _PAD + 1):
        hmin = jnp.minimum(hmin, padh[:, dx:dx + _W])
    return hmin


def _dc_kernel(img_ref, dc_ref):
    dc_ref[0] = lax.bitcast_convert_type(_dark_channel(img_ref[0]), jnp.int32)


def _sc_select_body(dc_hbm, img_hbm, out_hbm,
                    dc_v, img_v, cand_v, hist_v, merged_v, tmp4_v,
                    row_v, cnt_v, hist_sh, cnt_sh):
    cid = lax.axis_index("c")
    sid = lax.axis_index("s")
    img_local = sid // _GPI
    image_id = cid * (_NS // _GPI) + img_local
    g = sid % _GPI
    s0 = img_local * _GPI
    base = g * _CHUNK

    lane = lax.iota(jnp.int32, 16)
    ones = jnp.ones((_L,), jnp.int32)
    zeros16 = jnp.zeros((_L,), jnp.int32)

    rows0 = g * (_CHUNK // _W)
    pltpu.sync_copy(dc_hbm.at[image_id, pl.ds(rows0, _CHUNK // _W), :], dc_v)

    k_l = jnp.int32(_K)
    prefix = jnp.int32(0)
    b_l = jnp.int32(0)
    ncand = jnp.int32(0)
    for lvl, (shift, bits) in enumerate(_LEVELS):
        width = 1 << bits

        def zero_body(j, _):
            for u in range(8):
                hist_v[pl.ds((j * 8 + u) * _L, _L)] = zeros16
            return 0
        lax.fori_loop(0, width // 8, zero_body, 0)

        psh = shift + bits

        if lvl == 0:
            def hist0_body(r, _):
                for u in range(_W // _L):
                    b = dc_v[r, pl.ds(u * _L, _L)]
                    binv = lax.shift_right_logical(b, shift)
                    plsc.addupdate_scatter(hist_v, [lane * width + binv],
                                           ones)
                return 0
            lax.fori_loop(0, _CHUNK // _W, hist0_body, 0)
        else:
            ncv = (ncand + (_L - 1)) // _L

            def histc_body(i, _):
                b = cand_v[pl.ds(i * _L, _L)]
                binv = lax.shift_right_logical(b, shift) & (width - 1)
                valid = (i * _L + lane) < ncand
                msk = (lax.shift_right_logical(b, psh) == prefix) & valid
                plsc.addupdate_scatter(hist_v, [lane * width + binv], ones,
                                       mask=msk)
                return 0
            lax.fori_loop(0, ncv, histc_body, 0)

        def lmerge_body(j, _):
            acc = zeros16
            for l in range(_L):
                acc = acc + hist_v[pl.ds(l * width + j * _L, _L)]
            merged_v[pl.ds(j * _L, _L)] = acc
            return 0
        lax.fori_loop(0, width // _L, lmerge_body, 0)

        pltpu.sync_copy(merged_v.at[pl.ds(0, width)],
                        hist_sh.at[sid, pl.ds(lvl * 256, width)])
        plsc.subcore_barrier()
        for gg in range(_GPI):
            pltpu.sync_copy(hist_sh.at[s0 + gg, pl.ds(lvl * 256, width)],
                            tmp4_v.at[pl.ds(gg * width, width)])

        def gmerge_body(j, _):
            acc = zeros16
            for gg in range(_GPI):
                acc = acc + tmp4_v[pl.ds(gg * width + j * _L, _L)]
            merged_v[pl.ds(j * _L, _L)] = acc
            return 0
        lax.fori_loop(0, width // _L, gmerge_body, 0)

        nch = width // _L

        def scan_body(j, carry):
            above, nsat = carry
            jc = nch - 1 - j
            h = merged_v[pl.ds(jc * _L, _L)]
            suf = lax.rev(plsc.cumsum(lax.rev(h, (0,))), (0,)) + above
            nsat = nsat + jnp.where(suf >= k_l, 1, 0)
            above = above + jnp.sum(h)
            return (above, nsat)
        _, nsat_vec = lax.fori_loop(0, nch, scan_body,
                                    (jnp.int32(0), zeros16))
        b_l = jnp.sum(nsat_vec) - 1

        def above_body(j, acc):
            h = merged_v[pl.ds(j * _L, _L)]
            bin_ids = j * _L + lane
            return acc + jnp.where(bin_ids > b_l, h, 0)
        acc_vec = lax.fori_loop(0, nch, above_body, zeros16)
        a_l = jnp.sum(acc_vec)

        k_l = k_l - a_l
        prefix = (prefix << bits) | b_l

        if lvl == 0:
            sh0 = _LEVELS[0][0]

            def compact_body(r, ptr):
                for u in range(_W // _L):
                    b = dc_v[r, pl.ds(u * _L, _L)]
                    eqb = lax.shift_right_logical(b, sh0) == b_l
                    pc = plsc.cumsum(jnp.where(eqb, 1, 0))
                    idx = ptr + jnp.maximum(pc, 1) - 1
                    plsc.store_scatter(cand_v, [idx], b, mask=eqb)
                    ptr = ptr + plsc.all_reduce_population_count(eqb)
                return ptr
            ptr_vec = lax.fori_loop(0, _CHUNK // _W, compact_body, zeros16)
            ncand = jnp.max(ptr_vec)

    t_bits = prefix
    m = k_l

    last_w = 1 << _LEVELS[-1][1]
    my_ties = jnp.sum(plsc.load_gather(hist_v, [lane * last_w + b_l]))

    cnt_v[...] = jnp.full((_L,), my_ties, jnp.int32)
    pltpu.sync_copy(cnt_v, cnt_sh.at[sid])
    plsc.subcore_barrier()
    prior = jnp.int32(0)
    for gg in range(_GPI):
        pltpu.sync_copy(cnt_sh.at[s0 + gg], cnt_v)
        c_gg = jnp.max(cnt_v[...])
        prior = prior + jnp.where(jnp.int32(gg) < g, c_gg, 0)
    m_g = m - prior

    subr = _CHUNK // (_NSUB * _W)
    acc0 = jnp.full((_L,), -1.0, jnp.float32)
    run = zeros16
    accs = (acc0, acc0, acc0)
    for sch in range(_NSUB):
        for c in range(_C):
            pltpu.sync_copy(
                img_hbm.at[image_id, c, pl.ds(rows0 + sch * subr, subr), :],
                img_v.at[c])

        def fuse_body(r, carry):
            run, a0, a1, a2 = carry
            for u in range(_W // _L):
                b = dc_v[sch * subr + r, pl.ds(u * _L, _L)]
                gt = b > t_bits
                eq = b == t_bits
                pc = plsc.cumsum(jnp.where(eq, 1, 0))
                sel = gt | (eq & ((run + pc) <= m_g))
                run = run + plsc.all_reduce_population_count(eq)
                v0 = img_v[0, r, pl.ds(u * _L, _L)]
                v1 = img_v[1, r, pl.ds(u * _L, _L)]
                v2 = img_v[2, r, pl.ds(u * _L, _L)]
                a0 = jnp.maximum(a0, jnp.where(sel, v0, -1.0))
                a1 = jnp.maximum(a1, jnp.where(sel, v1, -1.0))
                a2 = jnp.maximum(a2, jnp.where(sel, v2, -1.0))
            return (run, a0, a1, a2)
        run, *accs = lax.fori_loop(0, subr, fuse_body, (run, *accs))

    for c in range(_C):
        row_v[pl.ds(c * _L, _L)] = accs[c]

    off = image_id * (_GPI * _C * _L) + g * (_C * _L)
    pltpu.sync_copy(row_v, out_hbm.at[pl.ds(off, _C * _L)])


_sc_select = functools.partial(
    pl.kernel,
    out_type=jax.ShapeDtypeStruct((_B * _GPI * _C * _L,), jnp.float32),
    mesh=plsc.VectorSubcoreMesh(core_axis_name="c", subcore_axis_name="s",
                                num_cores=_NC, num_subcores=_NS),
    compiler_params=pltpu.CompilerParams(needs_layout_passes=False),
    scratch_types=[
        pltpu.VMEM((_CHUNK // _W, _W), jnp.int32),
        pltpu.VMEM((_C, _CHUNK // (_NSUB * _W), _W), jnp.float32),
        pltpu.VMEM((_CHUNK,), jnp.int32),
        pltpu.VMEM((_L * 256,), jnp.int32),
        pltpu.VMEM((256,), jnp.int32),
        pltpu.VMEM((_GPI * 256,), jnp.int32),
        pltpu.VMEM((_C * _L,), jnp.float32),
        pltpu.VMEM((_L,), jnp.int32),
        pltpu.VMEM_SHARED((_NS, 4 * 256), jnp.int32),
        pltpu.VMEM_SHARED((_NS, _L), jnp.int32),
    ],
)(_sc_select_body)


def _combine_kernel(part_ref, out_ref):
    x = part_ref[...]
    total = jnp.float32(0.0)
    for c in range(_C):
        mc = jnp.full((_B, 1), -1.0, jnp.float32)
        for g in range(_GPI):
            blk = x[:, g * _C * _L + c * _L: g * _C * _L + (c + 1) * _L]
            mc = jnp.maximum(mc, jnp.max(blk, axis=1, keepdims=True))
        total = total + jnp.sum(jnp.minimum(mc, _CLIP))
    out_ref[...] = (total / (_B * _C))[None, None]


@jax.jit
def kernel(image):
    dc = pl.pallas_call(
        _dc_kernel,
        grid=(_B,),
        in_specs=[pl.BlockSpec((1, _C, _H, _W), lambda b: (b, 0, 0, 0))],
        out_specs=pl.BlockSpec((1, _H, _W), lambda b: (b, 0, 0)),
        out_shape=jax.ShapeDtypeStruct((_B, _H, _W), jnp.int32),
        compiler_params=pltpu.CompilerParams(
            dimension_semantics=("arbitrary",),
        ),
    )(image)
    partial = _sc_select(dc, image)
    out = pl.pallas_call(
        _combine_kernel,
        out_shape=jax.ShapeDtypeStruct((1, 1), jnp.float32),
    )(partial.reshape(_B, _GPI * _C * _L))
    return out.reshape(())


# speedup vs baseline: 1.5268x; 1.0005x over previous
import functools

import jax
import jax.numpy as jnp
from jax import lax
from jax.experimental import pallas as pl
from jax.experimental.pallas import tpu as pltpu
from jax.experimental.pallas import tpu_sc as plsc

_B, _C, _H, _W = 8, 3, 384, 384
_N = _H * _W
_K = 1327
_PAD = 3
_CLIP = 0.89

_NC, _NS, _L = 2, 16, 16
_GPI = 4
_CHUNK = _N // _GPI
_VPT = _CHUNK // _L
_LEVELS = ((23, 8), (15, 8), (7, 8), (0, 7))
_NSUB = 3


def _dark_channel(x):
    cmin = jnp.min(x, axis=0)
    inf_rows = jnp.full((_PAD, _W), jnp.inf, dtype=cmin.dtype)
    padv = jnp.concatenate([inf_rows, cmin, inf_rows], axis=0)
    vmin = padv[0:_H]
    for dy in range(1, 2 * _PAD + 1):
        vmin = jnp.minimum(vmin, padv[dy:dy + _H])
    inf_cols = jnp.full((_H, _PAD), jnp.inf, dtype=cmin.dtype)
    padh = jnp.concatenate([inf_cols, vmin, inf_cols], axis=1)
    hmin = padh[:, 0:_W]
    for dx in range(1, 2 * _PAD + 1):
        hmin = jnp.minimum(hmin, padh[:, dx:dx + _W])
    return hmin


def _dc_kernel(img_ref, dc_ref):
    dc_ref[0] = lax.bitcast_convert_type(_dark_channel(img_ref[0]), jnp.int32)


def _sc_select_body(dc_hbm, img_hbm, out_hbm,
                    dc_v, img_v, cand_v, hist_v, merged_v, tmp4_v,
                    row_v, cnt_v, hist_sh, cnt_sh):
    cid = lax.axis_index("c")
    sid = lax.axis_index("s")
    img_local = sid // _GPI
    image_id = cid * (_NS // _GPI) + img_local
    g = sid % _GPI
    s0 = img_local * _GPI

    lane = lax.iota(jnp.int32, 16)
    ones = jnp.ones((_L,), jnp.int32)
    zeros16 = jnp.zeros((_L,), jnp.int32)

    rows0 = g * (_CHUNK // _W)
    pltpu.sync_copy(dc_hbm.at[image_id, pl.ds(rows0, _CHUNK // _W), :], dc_v)

    k_l = jnp.int32(_K)
    prefix = jnp.int32(0)
    b_l = jnp.int32(0)
    ncand = jnp.int32(0)
    for lvl, (shift, bits) in enumerate(_LEVELS):
        width = 1 << bits

        def zero_body(j, _):
            for u in range(8):
                hist_v[pl.ds((j * 8 + u) * _L, _L)] = zeros16
            return 0
        lax.fori_loop(0, width // 8, zero_body, 0)

        psh = shift + bits

        if lvl == 0:
            def hist0_body(r, _):
                for u in range(_W // _L):
                    b = dc_v[r, pl.ds(u * _L, _L)]
                    binv = lax.shift_right_logical(b, shift)
                    plsc.addupdate_scatter(hist_v, [lane * width + binv],
                                           ones)
                return 0
            lax.fori_loop(0, _CHUNK // _W, hist0_body, 0)
        else:
            ncv = (ncand + (_L - 1)) // _L

            def histc_body(i, _):
                b = cand_v[pl.ds(i * _L, _L)]
                binv = lax.shift_right_logical(b, shift) & (width - 1)
                valid = (i * _L + lane) < ncand
                msk = (lax.shift_right_logical(b, psh) == prefix) & valid
                plsc.addupdate_scatter(hist_v, [lane * width + binv], ones,
                                       mask=msk)
                return 0
            lax.fori_loop(0, ncv, histc_body, 0)

        def lmerge_body(j, _):
            acc = zeros16
            for l in range(_L):
                acc = acc + hist_v[pl.ds(l * width + j * _L, _L)]
            merged_v[pl.ds(j * _L, _L)] = acc
            return 0
        lax.fori_loop(0, width // _L, lmerge_body, 0)

        pltpu.sync_copy(merged_v.at[pl.ds(0, width)],
                        hist_sh.at[sid, pl.ds(lvl * 256, width)])
        plsc.subcore_barrier()
        for gg in range(_GPI):
            pltpu.sync_copy(hist_sh.at[s0 + gg, pl.ds(lvl * 256, width)],
                            tmp4_v.at[pl.ds(gg * width, width)])

        def gmerge_body(j, _):
            acc = zeros16
            for gg in range(_GPI):
                acc = acc + tmp4_v[pl.ds(gg * width + j * _L, _L)]
            merged_v[pl.ds(j * _L, _L)] = acc
            return 0
        lax.fori_loop(0, width // _L, gmerge_body, 0)

        nch = width // _L

        def scan_body(j, carry):
            above, nsat = carry
            jc = nch - 1 - j
            h = merged_v[pl.ds(jc * _L, _L)]
            suf = lax.rev(plsc.cumsum(lax.rev(h, (0,))), (0,)) + above
            nsat = nsat + jnp.where(suf >= k_l, 1, 0)
            above = above + jnp.sum(h)
            return (above, nsat)
        _, nsat_vec = lax.fori_loop(0, nch, scan_body,
                                    (jnp.int32(0), zeros16))
        b_l = jnp.sum(nsat_vec) - 1

        def above_body(j, acc):
            h = merged_v[pl.ds(j * _L, _L)]
            bin_ids = j * _L + lane
            return acc + jnp.where(bin_ids > b_l, h, 0)
        acc_vec = lax.fori_loop(0, nch, above_body, zeros16)
        a_l = jnp.sum(acc_vec)

        k_l = k_l - a_l
        prefix = (prefix << bits) | b_l

        if lvl == 0:
            sh0 = _LEVELS[0][0]

            def compact_body(r, ptr):
                for u in range(_W // _L):
                    b = dc_v[r, pl.ds(u * _L, _L)]
                    eqb = lax.shift_right_logical(b, sh0) == b_l
                    pc = plsc.cumsum(jnp.where(eqb, 1, 0))
                    idx = ptr + jnp.maximum(pc, 1) - 1
                    plsc.store_scatter(cand_v, [idx], b, mask=eqb)
                    ptr = ptr + plsc.all_reduce_population_count(eqb)
                return ptr
            ptr_vec = lax.fori_loop(0, _CHUNK // _W, compact_body, zeros16)
            ncand = jnp.max(ptr_vec)

    t_bits = prefix
    m = k_l

    last_w = 1 << _LEVELS[-1][1]
    my_ties = jnp.sum(plsc.load_gather(hist_v, [lane * last_w + b_l]))

    cnt_v[...] = jnp.full((_L,), my_ties, jnp.int32)
    pltpu.sync_copy(cnt_v, cnt_sh.at[sid])
    plsc.subcore_barrier()
    prior = jnp.int32(0)
    for gg in range(_GPI):
        pltpu.sync_copy(cnt_sh.at[s0 + gg], cnt_v)
        c_gg = jnp.max(cnt_v[...])
        prior = prior + jnp.where(jnp.int32(gg) < g, c_gg, 0)
    m_g = m - prior

    subr = _CHUNK // (_NSUB * _W)
    acc0 = jnp.full((_L,), -1.0, jnp.float32)
    run = zeros16
    accs = (acc0, acc0, acc0)
    for sch in range(_NSUB):
        for c in range(_C):
            pltpu.sync_copy(
                img_hbm.at[image_id, c, pl.ds(rows0 + sch * subr, subr), :],
                img_v.at[c])

        def fuse_body(r, carry):
            run, a0, a1, a2 = carry
            for u in range(_W // _L):
                b = dc_v[sch * subr + r, pl.ds(u * _L, _L)]
                gt = b > t_bits
                eq = b == t_bits
                pc = plsc.cumsum(jnp.where(eq, 1, 0))
                sel = gt | (eq & ((run + pc) <= m_g))
                run = run + plsc.all_reduce_population_count(eq)
                v0 = img_v[0, r, pl.ds(u * _L, _L)]
                v1 = img_v[1, r, pl.ds(u * _L, _L)]
                v2 = img_v[2, r, pl.ds(u * _L, _L)]
                a0 = jnp.maximum(a0, jnp.where(sel, v0, -1.0))
                a1 = jnp.maximum(a1, jnp.where(sel, v1, -1.0))
                a2 = jnp.maximum(a2, jnp.where(sel, v2, -1.0))
            return (run, a0, a1, a2)
        run, *accs = lax.fori_loop(0, subr, fuse_body, (run, *accs))

    for c in range(_C):
        row_v[pl.ds(c * _L, _L)] = accs[c]

    off = image_id * (_GPI * _C * _L) + g * (_C * _L)
    pltpu.sync_copy(row_v, out_hbm.at[pl.ds(off, _C * _L)])


_sc_select = functools.partial(
    pl.kernel,
    out_type=jax.ShapeDtypeStruct((_B * _GPI * _C * _L,), jnp.float32),
    mesh=plsc.VectorSubcoreMesh(core_axis_name="c", subcore_axis_name="s",
                                num_cores=_NC, num_subcores=_NS),
    compiler_params=pltpu.CompilerParams(needs_layout_passes=False),
    scratch_types=[
        pltpu.VMEM((_CHUNK // _W, _W), jnp.int32),
        pltpu.VMEM((_C, _CHUNK // (_NSUB * _W), _W), jnp.float32),
        pltpu.VMEM((_CHUNK,), jnp.int32),
        pltpu.VMEM((_L * 256,), jnp.int32),
        pltpu.VMEM((256,), jnp.int32),
        pltpu.VMEM((_GPI * 256,), jnp.int32),
        pltpu.VMEM((_C * _L,), jnp.float32),
        pltpu.VMEM((_L,), jnp.int32),
        pltpu.VMEM_SHARED((_NS, 4 * 256), jnp.int32),
        pltpu.VMEM_SHARED((_NS, _L), jnp.int32),
    ],
)(_sc_select_body)


def _combine_kernel(part_ref, out_ref):
    x = part_ref[...]
    total = jnp.float32(0.0)
    for c in range(_C):
        mc = jnp.full((_B, 1), -1.0, jnp.float32)
        for g in range(_GPI):
            blk = x[:, g * _C * _L + c * _L: g * _C * _L + (c + 1) * _L]
            mc = jnp.maximum(mc, jnp.max(blk, axis=1, keepdims=True))
        total = total + jnp.sum(jnp.minimum(mc, _CLIP))
    out_ref[...] = (total / (_B * _C))[None, None]


@jax.jit
def kernel(image):
    dc = pl.pallas_call(
        _dc_kernel,
        grid=(_B,),
        in_specs=[pl.BlockSpec((1, _C, _H, _W), lambda b: (b, 0, 0, 0))],
        out_specs=pl.BlockSpec((1, _H, _W), lambda b: (b, 0, 0)),
        out_shape=jax.ShapeDtypeStruct((_B, _H, _W), jnp.int32),
        compiler_params=pltpu.CompilerParams(
            dimension_semantics=("arbitrary",),
        ),
    )(image)
    partial = _sc_select(dc, image)
    out = pl.pallas_call(
        _combine_kernel,
        out_shape=jax.ShapeDtypeStruct((1, 1), jnp.float32),
    )(partial.reshape(_B, _GPI * _C * _L))
    return out.reshape(())
